# Initial kernel scaffold; baseline (speedup 1.0000x reference)
import jax
import jax.numpy as jnp
from jax.experimental import pallas as pl


def kernel(x, scale_idx, scale_embeddings, router_W, W1, b1, W2, b2):
    raise NotImplementedError("write your pallas kernel here")


# speedup vs baseline: 2.0718x; 2.0718x over previous
import functools

import jax
import jax.numpy as jnp
from jax.experimental import pallas as pl
from jax.experimental.pallas import tpu as pltpu


def _router_body(x_ref, semb_ref, rw_x_ref, rw_s_ref, disp_ref):
    x = x_ref[...]
    logits = jax.lax.dot_general(
        x, rw_x_ref[...], (((1,), (0,)), ((), ())),
        preferred_element_type=jnp.float32)
    logits += jax.lax.dot_general(
        semb_ref[...], rw_s_ref[...], (((1,), (0,)), ((), ())),
        preferred_element_type=jnp.float32)
    probs = jax.nn.softmax(logits, axis=-1)
    e = probs.shape[-1]
    iota = jax.lax.broadcasted_iota(jnp.int32, probs.shape, 1)
    big = jnp.int32(e + 1)
    m1 = jnp.max(probs, axis=-1, keepdims=True)
    am1 = jnp.min(jnp.where(probs == m1, iota, big), axis=-1, keepdims=True)
    probs2 = jnp.where(iota == am1, -jnp.inf, probs)
    m2 = jnp.max(probs2, axis=-1, keepdims=True)
    am2 = jnp.min(jnp.where(probs2 == m2, iota, big), axis=-1, keepdims=True)
    sel = (iota == am1) | (iota == am2)
    w = jnp.where(sel, probs, 0.0)
    disp_ref[...] = w / jnp.sum(w, axis=-1, keepdims=True)


def _ffn_body(disp_ref, x_ref, w1_ref, b1_ref, w2_ref, b2_ref, out_ref,
              acc_ref, *, n_experts, tb):
    e = pl.program_id(0)
    t = pl.program_id(1)
    rows = pl.ds(t * tb, tb)

    @pl.when(e == 0)
    def _init():
        acc_ref[rows, :] = jnp.zeros_like(out_ref)

    x = x_ref[...]
    h = jax.lax.dot_general(
        x, w1_ref[0], (((1,), (0,)), ((), ())),
        preferred_element_type=jnp.float32)
    h += b1_ref[0].astype(jnp.float32)
    h = 0.5 * h * (1.0 + jax.lax.erf(h * 0.7071067811865476))
    y = jax.lax.dot_general(
        h.astype(jnp.bfloat16), w2_ref[0], (((1,), (0,)), ((), ())),
        preferred_element_type=jnp.float32)
    y += b2_ref[0].astype(jnp.float32)
    disp = disp_ref[...]
    eiota = jax.lax.broadcasted_iota(jnp.int32, disp.shape, 1)
    w = jnp.sum(jnp.where(eiota == e, disp, 0.0), axis=1, keepdims=True)
    acc_ref[rows, :] += y * w

    @pl.when(e == n_experts - 1)
    def _flush():
        out_ref[...] = acc_ref[rows, :]


def kernel(x, scale_idx, scale_embeddings, router_W, W1, b1, W2, b2):
    b, s, d = x.shape
    n_experts, _, hidden = W1.shape
    se = scale_embeddings.shape[-1]
    x2 = x.reshape(s, d)
    scale_emb = jax.lax.dynamic_slice_in_dim(
        scale_embeddings, scale_idx, 1, axis=0)

    tb = 256 if s % 256 == 0 else s
    n_tb = s // tb

    dispatch = pl.pallas_call(
        _router_body,
        grid=(n_tb,),
        in_specs=[
            pl.BlockSpec((tb, d), lambda t: (t, 0)),
            pl.BlockSpec((1, se), lambda t: (0, 0)),
            pl.BlockSpec((d, n_experts), lambda t: (0, 0)),
            pl.BlockSpec((se, n_experts), lambda t: (0, 0)),
        ],
        out_specs=pl.BlockSpec((tb, n_experts), lambda t: (t, 0)),
        out_shape=jax.ShapeDtypeStruct((s, n_experts), jnp.float32),
    )(x2, scale_emb, router_W[:d], router_W[d:])

    x_bf = x2.astype(jnp.bfloat16)
    w1_bf = W1.astype(jnp.bfloat16)
    w2_bf = W2.astype(jnp.bfloat16)

    out = pl.pallas_call(
        functools.partial(_ffn_body, n_experts=n_experts, tb=tb),
        grid=(n_experts, n_tb),
        in_specs=[
            pl.BlockSpec((tb, n_experts), lambda e, t: (t, 0)),
            pl.BlockSpec((tb, d), lambda e, t: (t, 0)),
            pl.BlockSpec((1, d, hidden), lambda e, t: (e, 0, 0)),
            pl.BlockSpec((1, 1, hidden), lambda e, t: (e, 0, 0)),
            pl.BlockSpec((1, hidden, d), lambda e, t: (e, 0, 0)),
            pl.BlockSpec((1, 1, d), lambda e, t: (e, 0, 0)),
        ],
        out_specs=pl.BlockSpec((tb, d), lambda e, t: (t, 0)),
        out_shape=jax.ShapeDtypeStruct((s, d), jnp.float32),
        scratch_shapes=[pltpu.VMEM((s, d), jnp.float32)],
    )(dispatch, x_bf, w1_bf, b1.reshape(n_experts, 1, hidden),
      w2_bf, b2.reshape(n_experts, 1, d))

    return out.reshape(b, s, d)
